# Initial kernel scaffold; baseline (speedup 1.0000x reference)
import jax
import jax.numpy as jnp
from jax.experimental import pallas as pl


def kernel(x, edge_index, W1, b1, W2, b2):
    raise NotImplementedError("write your pallas kernel here")


# speedup vs baseline: 7.9514x; 7.9514x over previous
import functools

import jax
import jax.numpy as jnp
from jax import lax
from jax.experimental import pallas as pl
from jax.experimental.pallas import tpu as pltpu
from jax.experimental.pallas import tpu_sc as plsc

_N = 10000
_E = 320000
_D_IN = 128
_D_HID = 256
_D_OUT = 128

_NC = 2
_NS = 16
_MC = 128
_E_PAD = 327680
_NCHUNK = _E_PAD // _MC
_NP = 10240
_NPT = _NP // _NS


@functools.cache
def _agg_call(chunks_per_tile, off_mult, core_stride, gather=True):
    mesh = plsc.VectorSubcoreMesh(core_axis_name="c", subcore_axis_name="s")
    group = 16
    ngroups = chunks_per_tile // group
    assert chunks_per_tile % group == 0

    @functools.partial(
        pl.kernel,
        out_type=jax.ShapeDtypeStruct((2 * _NP, 128), jnp.float32),
        mesh=mesh,
        scratch_types=[
            pltpu.VMEM((group, _MC), jnp.int32),
            pltpu.VMEM((group, _MC), jnp.int32),
            pltpu.VMEM((_MC, 128), jnp.float32),
            pltpu.SemaphoreType.DMA,
            pltpu.VMEM_SHARED((_NP, 128), jnp.float32),
        ],
    )
    def agg_kernel(table_hbm, src_hbm, dst_hbm, out_hbm, sidx, didx, rows, sem, acc):
        c = lax.axis_index("c")
        s = lax.axis_index("s")
        row0 = s * _NPT
        for t in range(_NPT // _MC):
            pltpu.sync_copy(
                table_hbm.at[pl.ds(c * off_mult + row0 + t * _MC, _MC)], rows)
            pltpu.sync_copy(rows, acc.at[pl.ds(row0 + t * _MC, _MC)])
        plsc.subcore_barrier()
        if not gather:
            pltpu.sync_copy(table_hbm.at[pl.ds(0, _MC)], rows)
        base = (c * core_stride + s) * chunks_per_tile
        goff = jnp.full((16,), c * off_mult, jnp.int32)

        def grp(g, _):
            gbase = base + g * group
            if gather:
                pltpu.sync_copy(src_hbm.at[pl.ds(gbase, group)], sidx)
            pltpu.sync_copy(dst_hbm.at[pl.ds(gbase, group)], didx)
            if gather and off_mult:
                def addoff(j, _):
                    for l in range(_MC // 16):
                        sidx[j, pl.ds(l * 16, 16)] = sidx[j, pl.ds(l * 16, 16)] + goff
                    return 0

                lax.fori_loop(0, group, addoff, 0)

            for j in range(group):
                if gather:
                    pltpu.async_copy(table_hbm.at[sidx.at[j]], rows, sem).wait()
                pltpu.sync_copy(rows, acc.at[didx.at[j]], add=True)
            return 0

        lax.fori_loop(0, ngroups, grp, 0)
        plsc.subcore_barrier()
        for t in range(_NPT // _MC):
            pltpu.sync_copy(acc.at[pl.ds(row0 + t * _MC, _MC)], rows)
            pltpu.sync_copy(rows, out_hbm.at[pl.ds(c * _NP + row0 + t * _MC, _MC)])

    return agg_kernel


_R = 1024
_NB = _NP // _R


def _scale1(x, W1, deg):
    def body(x_ref, w_ref, dega_ref, degb_ref, out_ref):
        dis = lax.rsqrt(dega_ref[:, 0:1] + degb_ref[:, 0:1] - 1.0)
        out_ref[:, :] = dis * jnp.dot(x_ref[:, :], w_ref[:, :],
                                      preferred_element_type=jnp.float32)

    return pl.pallas_call(
        body,
        grid=(2, _NB),
        in_specs=[
            pl.BlockSpec((_R, _D_IN), lambda h, i: (i, 0)),
            pl.BlockSpec((_D_IN, 128), lambda h, i: (0, h)),
            pl.BlockSpec((_R, 128), lambda h, i: (i, 0)),
            pl.BlockSpec((_R, 128), lambda h, i: (_NB + i, 0)),
        ],
        out_specs=pl.BlockSpec((_R, 128), lambda h, i: (h * _NB + i, 0)),
        out_shape=jax.ShapeDtypeStruct((2 * _NP, 128), jnp.float32),
    )(x, W1, deg, deg)


def _combine1(S1, deg, b1, W2):
    def body(sa_ref, sb_ref, dega_ref, degb_ref, b1_ref, w2_ref, out_ref):
        dis = lax.rsqrt(dega_ref[:, 0:1] + degb_ref[:, 0:1] - 1.0)
        za = dis * sa_ref[:, :] + b1_ref[0:1, 0:128]
        zb = dis * sb_ref[:, :] + b1_ref[0:1, 128:256]
        a = jnp.concatenate([jnp.maximum(za, 0.0), jnp.maximum(zb, 0.0)], axis=1)
        out_ref[:, :] = dis * jnp.dot(a, w2_ref[:, :],
                                      preferred_element_type=jnp.float32)

    return pl.pallas_call(
        body,
        grid=(_NB,),
        in_specs=[
            pl.BlockSpec((_R, 128), lambda i: (i, 0)),
            pl.BlockSpec((_R, 128), lambda i: (_NB + i, 0)),
            pl.BlockSpec((_R, 128), lambda i: (i, 0)),
            pl.BlockSpec((_R, 128), lambda i: (_NB + i, 0)),
            pl.BlockSpec((1, _D_HID), lambda i: (0, 0)),
            pl.BlockSpec((_D_HID, _D_OUT), lambda i: (0, 0)),
        ],
        out_specs=pl.BlockSpec((_R, _D_OUT), lambda i: (i, 0)),
        out_shape=jax.ShapeDtypeStruct((_NP, _D_OUT), jnp.float32),
    )(S1, S1, deg, deg, b1.reshape(1, _D_HID), W2)


def _final(S2, hp2, deg, b2):
    def body(sa_ref, sb_ref, hp_ref, dega_ref, degb_ref, b2_ref, out_ref):
        dis = lax.rsqrt(dega_ref[:, 0:1] + degb_ref[:, 0:1] - 1.0)
        h = dis * (sa_ref[:, :] + sb_ref[:, :] - hp_ref[:, :]) + b2_ref[0:1, :]
        nrm = jnp.sqrt(jnp.sum(h * h, axis=1, keepdims=True))
        out_ref[:, :] = h / jnp.maximum(nrm, 1e-12)

    return pl.pallas_call(
        body,
        grid=(_NB,),
        in_specs=[
            pl.BlockSpec((_R, 128), lambda i: (i, 0)),
            pl.BlockSpec((_R, 128), lambda i: (_NB + i, 0)),
            pl.BlockSpec((_R, 128), lambda i: (i, 0)),
            pl.BlockSpec((_R, 128), lambda i: (i, 0)),
            pl.BlockSpec((_R, 128), lambda i: (_NB + i, 0)),
            pl.BlockSpec((1, _D_OUT), lambda i: (0, 0)),
        ],
        out_specs=pl.BlockSpec((_R, _D_OUT), lambda i: (i, 0)),
        out_shape=jax.ShapeDtypeStruct((_N, _D_OUT), jnp.float32),
    )(S2, S2, hp2, deg, deg, b2.reshape(1, _D_OUT))


def kernel(x, edge_index, W1, b1, W2, b2):
    src = edge_index[0]
    dst = edge_index[1]
    npad = _E_PAD - _E
    src2d = jnp.concatenate(
        [src, jnp.zeros((npad,), jnp.int32)]).reshape(_NCHUNK, _MC)
    dst2d = jnp.concatenate(
        [dst, jnp.full((npad,), _N, jnp.int32)]).reshape(_NCHUNK, _MC)

    ones_tab = jnp.ones((_NP, 128), jnp.float32)
    deg = _agg_call(_NCHUNK // (_NC * _NS), 0, _NS, False)(ones_tab, src2d, dst2d)
    hp1 = _scale1(x, W1, deg)
    S1 = _agg_call(_NCHUNK // _NS, _NP, 0)(hp1, src2d, dst2d)
    hp2 = _combine1(S1, deg, b1, W2)
    S2 = _agg_call(_NCHUNK // (_NC * _NS), 0, _NS)(hp2, src2d, dst2d)
    return _final(S2, hp2, deg, b2)
